# baseline (speedup 1.0000x reference)
import functools

import jax
import jax.numpy as jnp
from jax import lax
from jax.experimental import pallas as pl
from jax.experimental.pallas import tpu as pltpu
from jax.experimental.pallas import tpu_sc as plsc


@functools.lru_cache(maxsize=None)
def _build_lookup(B, NBITS, V, D):
    info = plsc.get_sparse_core_info()
    NC, NS, L = info.num_cores, info.num_subcores, info.num_lanes
    NW = NC * NS
    assert B % NW == 0
    b_per_w = B // NW
    GCHUNK = 128
    G = b_per_w // GCHUNK
    mesh = plsc.VectorSubcoreMesh(core_axis_name="c", subcore_axis_name="s")

    @functools.partial(
        pl.kernel,
        mesh=mesh,
        out_type=jax.ShapeDtypeStruct((B, D), jnp.float32),
        compiler_params=pltpu.CompilerParams(use_tc_tiling_on_sc=False),
        scratch_types=[
            pltpu.VMEM((NBITS, b_per_w), jnp.int32),
            pltpu.VMEM((G, GCHUNK), jnp.int32),
            pltpu.VMEM((b_per_w, D), jnp.float32),
            pltpu.SemaphoreType.DMA,
        ],
    )
    def lookup(bits_hbm, table_hbm, out_hbm, bits_v, idx_v, rows_v, sem):
        wid = lax.axis_index("s") * NC + lax.axis_index("c")
        base = wid * b_per_w

        stage = [
            pltpu.async_copy(
                bits_hbm.at[pl.ds(j * B + base, b_per_w)], bits_v.at[j], sem
            )
            for j in range(NBITS)
        ]
        for cp in stage:
            cp.wait()

        for i in range(b_per_w // L):
            v = bits_v[0, pl.ds(i * L, L)]
            for j in range(1, NBITS):
                v = v + v + bits_v[j, pl.ds(i * L, L)]
            idx_v[i // (GCHUNK // L), pl.ds((i % (GCHUNK // L)) * L, L)] = v

        gather = [
            pltpu.async_copy(
                table_hbm.at[idx_v.at[g]],
                rows_v.at[pl.ds(g * GCHUNK, GCHUNK)],
                sem,
            )
            for g in range(G)
        ]
        for cp in gather:
            cp.wait()

        pltpu.sync_copy(rows_v, out_hbm.at[pl.ds(base, b_per_w)])

    return lookup


def kernel(input_ids, table):
    B, NBITS = input_ids.shape
    V, D = table.shape
    bits_t = input_ids.T.reshape(NBITS * B)
    return _build_lookup(B, NBITS, V, D)(bits_t, table)


# speedup vs baseline: 7.0441x; 7.0441x over previous
import functools

import jax
import jax.numpy as jnp
from jax import lax
from jax.experimental import pallas as pl
from jax.experimental.pallas import tpu as pltpu
from jax.experimental.pallas import tpu_sc as plsc


@functools.lru_cache(maxsize=None)
def _build_lookup(B, NBITS, V, D):
    info = plsc.get_sparse_core_info()
    NC, NS, L = info.num_cores, info.num_subcores, info.num_lanes
    NW = NC * NS
    assert B % NW == 0
    b_per_w = B // NW
    e_per_w = b_per_w * D
    GCH = 128
    NST = e_per_w // GCH
    OROW = e_per_w // 128
    mesh = plsc.VectorSubcoreMesh(core_axis_name="c", subcore_axis_name="s")

    @functools.partial(
        pl.kernel,
        mesh=mesh,
        out_type=jax.ShapeDtypeStruct((B * D // 128, 128), jnp.float32),
        compiler_params=pltpu.CompilerParams(
            use_tc_tiling_on_sc=False, needs_layout_passes=False
        ),
        scratch_types=[
            pltpu.VMEM((NBITS, b_per_w), jnp.int32),
            pltpu.VMEM((b_per_w,), jnp.int32),
            pltpu.VMEM((NST, GCH), jnp.int32),
            pltpu.VMEM((OROW, 128), jnp.float32),
            pltpu.SemaphoreType.DMA,
            pltpu.SemaphoreType.DMA,
        ],
    )
    def lookup(bits_hbm, wflat_hbm, out_hbm, bits_v, p_v, eidx_v,
               gath_v, sem, sem2):
        wid = lax.axis_index("s") * NC + lax.axis_index("c")
        base = wid * b_per_w

        stage = [
            pltpu.async_copy(
                bits_hbm.at[pl.ds(j * B + base, b_per_w)], bits_v.at[j], sem
            )
            for j in range(NBITS)
        ]
        for cp in stage:
            cp.wait()

        @pl.loop(0, b_per_w // L)
        def _convert(i):
            v = bits_v[0, pl.ds(i * L, L)]
            for j in range(1, NBITS):
                v = v + v + bits_v[j, pl.ds(i * L, L)]
            p_v[pl.ds(i * L, L)] = (v >> 7) * 1024 + (v & 127)

        ktop = lax.iota(jnp.int32, L)
        off_lo = (ktop >> 3) * 8388608 + (ktop & 7) * 128
        ktop2 = ktop + L
        off_hi = (ktop2 >> 3) * 8388608 + (ktop2 & 7) * 128

        @pl.loop(0, b_per_w // L)
        def _build(blk):
            for t in range(L):
                pv = plsc.load_gather(
                    p_v, [jnp.full((L,), blk * L + t, jnp.int32)]
                )
                q = (blk * L + t) * D
                eidx_v[q // GCH, pl.ds(q % GCH, L)] = pv + off_lo
                q2 = q + L
                eidx_v[q2 // GCH, pl.ds(q2 % GCH, L)] = pv + off_hi

        @pl.loop(0, NST // 8)
        def _gather(jb):
            cps = [
                pltpu.async_copy(
                    wflat_hbm.at[eidx_v.at[jb * 8 + t]],
                    gath_v.at[jb * 8 + t],
                    sem2,
                )
                for t in range(8)
            ]
            for cp in cps:
                cp.wait()

        pltpu.sync_copy(gath_v, out_hbm.at[pl.ds(wid * OROW, OROW)])

    return lookup


def kernel(input_ids, table):
    B, NBITS = input_ids.shape
    V, D = table.shape
    bits_t = input_ids.T.reshape(NBITS * B)
    wflat = (
        table.T.reshape(D // 8, 8, V // 128, 128)
        .transpose(0, 2, 1, 3)
        .reshape(V * D)
    )
    out = _build_lookup(B, NBITS, V, D)(bits_t, wflat)
    return out.reshape(B, D)


# speedup vs baseline: 10.9690x; 1.5572x over previous
import functools

import jax
import jax.numpy as jnp
from jax import lax
from jax.experimental import pallas as pl
from jax.experimental.pallas import tpu as pltpu
from jax.experimental.pallas import tpu_sc as plsc


@functools.lru_cache(maxsize=None)
def _build_lookup(B, NBITS, V, D):
    info = plsc.get_sparse_core_info()
    NC, NS, L = info.num_cores, info.num_subcores, info.num_lanes
    NW = NC * NS
    assert B % (NW * 128) == 0 and D % 8 == 0 and V % 128 == 0
    b_per_w = B // NW
    e_per_w = b_per_w * D
    TCL = b_per_w // 128
    NROW = e_per_w // 128
    TBLK = V // 128
    OBLK = B // 128
    mesh = plsc.VectorSubcoreMesh(core_axis_name="c", subcore_axis_name="s")

    @functools.partial(
        pl.kernel,
        mesh=mesh,
        out_type=jax.ShapeDtypeStruct((B * D // 128, 128), jnp.float32),
        compiler_params=pltpu.CompilerParams(
            use_tc_tiling_on_sc=False, needs_layout_passes=False
        ),
        scratch_types=[
            pltpu.VMEM((NBITS, b_per_w), jnp.int32),
            pltpu.VMEM((b_per_w,), jnp.int32),
            pltpu.VMEM((e_per_w,), jnp.int32),
            pltpu.VMEM((e_per_w,), jnp.float32),
            pltpu.SemaphoreType.DMA,
            pltpu.SemaphoreType.DMA,
        ],
    )
    def lookup(bits_hbm, wflat_hbm, out_hbm, bits_v, p_v, eidx_v, gath_v,
               sem, sem2):
        wid = lax.axis_index("s") * NC + lax.axis_index("c")
        base = wid * b_per_w

        stage = [
            pltpu.async_copy(
                bits_hbm.at[pl.ds(j * B + base, b_per_w)], bits_v.at[j], sem
            )
            for j in range(NBITS)
        ]
        for cp in stage:
            cp.wait()

        @pl.loop(0, b_per_w // L)
        def _convert(i):
            v = bits_v[0, pl.ds(i * L, L)]
            for j in range(1, NBITS):
                v = v + v + bits_v[j, pl.ds(i * L, L)]
            p_v[pl.ds(i * L, L)] = (v >> 7) * 1024 + (v & 127)

        for tcl in range(TCL):
            p_chunks = [
                p_v[pl.ds(tcl * 128 + ch * L, L)] for ch in range(128 // L)
            ]

            @pl.loop(0, D)
            def _build(rc):
                off = (rc >> 3) * (TBLK * 1024) + (rc & 7) * 128
                offv = jnp.full((L,), off, jnp.int32)
                qbase = (rc * TCL + tcl) * 128
                for ch in range(128 // L):
                    eidx_v[pl.ds(qbase + ch * L, L)] = offv + p_chunks[ch]

        pltpu.async_copy(wflat_hbm.at[eidx_v], gath_v, sem2).wait()

        out_cps = []
        for rc in range(D):
            for tcl in range(TCL):
                r = (rc >> 3) * (OBLK * 8) + (wid * TCL + tcl) * 8 + (rc & 7)
                out_cps.append(
                    pltpu.async_copy(
                        gath_v.at[pl.ds((rc * TCL + tcl) * 128, 128)],
                        out_hbm.at[r],
                        sem,
                    )
                )
        for cp in out_cps:
            cp.wait()

    return lookup


def kernel(input_ids, table):
    B, NBITS = input_ids.shape
    V, D = table.shape
    bits_t = input_ids.T.reshape(NBITS * B)
    wflat = (
        table.T.reshape(D // 8, 8, V // 128, 128)
        .transpose(0, 2, 1, 3)
        .reshape(V * D)
    )
    out = _build_lookup(B, NBITS, V, D)(bits_t, wflat)
    out = (
        out.reshape(D // 8, B // 128, 8, 128)
        .transpose(0, 2, 1, 3)
        .reshape(D, B)
        .T
    )
    return out


# speedup vs baseline: 11.0563x; 1.0080x over previous
import functools

import jax
import jax.numpy as jnp
from jax import lax
from jax.experimental import pallas as pl
from jax.experimental.pallas import tpu as pltpu
from jax.experimental.pallas import tpu_sc as plsc


@functools.lru_cache(maxsize=None)
def _build_lookup(B, NBITS, V, D):
    info = plsc.get_sparse_core_info()
    NC, NS, L = info.num_cores, info.num_subcores, info.num_lanes
    NW = NC * NS
    assert B % (NW * 128) == 0 and D % 8 == 0 and V % 128 == 0
    b_per_w = B // NW
    e_per_w = b_per_w * D
    TCL = b_per_w // 128
    NROW = e_per_w // 128
    TBLK = V // 128
    OBLK = B // 128
    mesh = plsc.VectorSubcoreMesh(core_axis_name="c", subcore_axis_name="s")

    @functools.partial(
        pl.kernel,
        mesh=mesh,
        out_type=jax.ShapeDtypeStruct((B * D // 128, 128), jnp.float32),
        compiler_params=pltpu.CompilerParams(
            use_tc_tiling_on_sc=False, needs_layout_passes=False
        ),
        scratch_types=[
            pltpu.VMEM((NBITS, b_per_w), jnp.int32),
            pltpu.VMEM((b_per_w,), jnp.int32),
            pltpu.VMEM((e_per_w,), jnp.int32),
            pltpu.VMEM((e_per_w,), jnp.float32),
            pltpu.SemaphoreType.DMA,
            pltpu.SemaphoreType.DMA,
            pltpu.SemaphoreType.DMA,
            pltpu.SemaphoreType.DMA,
            pltpu.SemaphoreType.DMA,
        ],
    )
    def lookup(bits_hbm, wflat_hbm, out_hbm, bits_v, p_v, eidx_v, gath_v,
               sem, g0, g1, g2, g3):
        wid = lax.axis_index("s") * NC + lax.axis_index("c")
        base = wid * b_per_w
        gsems = [g0, g1, g2, g3]
        q_elems = e_per_w // TCL

        stage = [
            pltpu.async_copy(
                bits_hbm.at[pl.ds(j * B + base, b_per_w)], bits_v.at[j], sem
            )
            for j in range(NBITS)
        ]
        for cp in stage:
            cp.wait()

        gcps = []
        for tcl in range(TCL):
            @pl.loop(tcl * (128 // L), (tcl + 1) * (128 // L))
            def _convert(i):
                v = bits_v[0, pl.ds(i * L, L)]
                for j in range(1, NBITS):
                    v = v + v + bits_v[j, pl.ds(i * L, L)]
                p_v[pl.ds(i * L, L)] = (v >> 7) * 1024 + (v & 127)

            p_chunks = [
                p_v[pl.ds(tcl * 128 + ch * L, L)] for ch in range(128 // L)
            ]

            @pl.loop(0, D)
            def _build(rc):
                off = (rc >> 3) * (TBLK * 1024) + (rc & 7) * 128
                offv = jnp.full((L,), off, jnp.int32)
                qbase = (tcl * D + rc) * 128
                for ch in range(128 // L):
                    eidx_v[pl.ds(qbase + ch * L, L)] = offv + p_chunks[ch]

            gcps.append(
                pltpu.async_copy(
                    wflat_hbm.at[eidx_v.at[pl.ds(tcl * q_elems, q_elems)]],
                    gath_v.at[pl.ds(tcl * q_elems, q_elems)],
                    gsems[tcl],
                )
            )

        out_cps = []
        for tcl in range(TCL):
            gcps[tcl].wait()
            for rc in range(D):
                r = (rc >> 3) * (OBLK * 8) + (wid * TCL + tcl) * 8 + (rc & 7)
                out_cps.append(
                    pltpu.async_copy(
                        gath_v.at[pl.ds((tcl * D + rc) * 128, 128)],
                        out_hbm.at[r],
                        sem,
                    )
                )
        for cp in out_cps:
            cp.wait()

    return lookup


def kernel(input_ids, table):
    B, NBITS = input_ids.shape
    V, D = table.shape
    bits_t = input_ids.T.reshape(NBITS * B)
    wflat = (
        table.T.reshape(D // 8, 8, V // 128, 128)
        .transpose(0, 2, 1, 3)
        .reshape(V * D)
    )
    out = _build_lookup(B, NBITS, V, D)(bits_t, wflat)
    out = (
        out.reshape(D // 8, B // 128, 8, 128)
        .transpose(0, 2, 1, 3)
        .reshape(D, B)
        .T
    )
    return out


# speedup vs baseline: 11.5467x; 1.0444x over previous
import functools

import jax
import jax.numpy as jnp
from jax import lax
from jax.experimental import pallas as pl
from jax.experimental.pallas import tpu as pltpu
from jax.experimental.pallas import tpu_sc as plsc


@functools.lru_cache(maxsize=None)
def _build_lookup(B, NBITS, V, D):
    info = plsc.get_sparse_core_info()
    NC, NS, L = info.num_cores, info.num_subcores, info.num_lanes
    NW = NC * NS
    assert B % (NW * 128) == 0 and D % 8 == 0 and V % 128 == 0
    b_per_w = B // NW
    e_per_w = b_per_w * D
    TCL = b_per_w // 128
    NROW = e_per_w // 128
    TBLK = V // 128
    OBLK = B // 128
    mesh = plsc.VectorSubcoreMesh(core_axis_name="c", subcore_axis_name="s")

    @functools.partial(
        pl.kernel,
        mesh=mesh,
        out_type=jax.ShapeDtypeStruct((B * D,), jnp.float32),
        compiler_params=pltpu.CompilerParams(
            use_tc_tiling_on_sc=False, needs_layout_passes=False
        ),
        scratch_types=[
            pltpu.VMEM((NBITS, b_per_w), jnp.int32),
            pltpu.VMEM((b_per_w,), jnp.int32),
            pltpu.VMEM((e_per_w,), jnp.int32),
            pltpu.VMEM((e_per_w,), jnp.float32),
            pltpu.SemaphoreType.DMA,
            pltpu.SemaphoreType.DMA,
            pltpu.SemaphoreType.DMA,
            pltpu.SemaphoreType.DMA,
            pltpu.SemaphoreType.DMA,
        ],
    )
    def lookup(bits_hbm, wflat_hbm, out_hbm, bits_v, p_v, eidx_v, gath_v,
               sem, g0, g1, g2, g3):
        wid = lax.axis_index("s") * NC + lax.axis_index("c")
        base = wid * b_per_w
        gsems = [g0, g1, g2, g3]
        q_elems = e_per_w // TCL

        pltpu.sync_copy(bits_hbm.at[:, pl.ds(base, b_per_w)], bits_v)

        gcps = []
        for tcl in range(TCL):
            @pl.loop(tcl * (128 // L), (tcl + 1) * (128 // L))
            def _convert(i):
                v = bits_v[0, pl.ds(i * L, L)]
                for j in range(1, NBITS):
                    v = v + v + bits_v[j, pl.ds(i * L, L)]
                p_v[pl.ds(i * L, L)] = (v >> 7) * 1024 + (v & 127)

            p_chunks = [
                p_v[pl.ds(tcl * 128 + ch * L, L)] for ch in range(128 // L)
            ]

            @pl.loop(0, D)
            def _build(rc):
                off = (rc >> 3) * (TBLK * 1024) + (rc & 7) * 128
                offv = jnp.full((L,), off, jnp.int32)
                qbase = (tcl * D + rc) * 128
                for ch in range(128 // L):
                    eidx_v[pl.ds(qbase + ch * L, L)] = offv + p_chunks[ch]

            gcps.append(
                pltpu.async_copy(
                    wflat_hbm.at[eidx_v.at[pl.ds(tcl * q_elems, q_elems)]],
                    gath_v.at[pl.ds(tcl * q_elems, q_elems)],
                    gsems[tcl],
                )
            )

        out_cps = []
        for tcl in range(TCL):
            gcps[tcl].wait()
            for tr in range(D // 8):
                dst = (tr * OBLK + wid * TCL + tcl) * 8 * 128
                out_cps.append(
                    pltpu.async_copy(
                        gath_v.at[pl.ds((tcl * D + tr * 8) * 128, 8 * 128)],
                        out_hbm.at[pl.ds(dst, 8 * 128)],
                        sem,
                    )
                )
        for cp in out_cps:
            cp.wait()

    return lookup


def kernel(input_ids, table):
    B, NBITS = input_ids.shape
    V, D = table.shape
    bits_t = input_ids.T
    wflat = (
        table.T.reshape(D // 8, 8, V // 128, 128)
        .transpose(0, 2, 1, 3)
        .reshape(V * D)
    )
    out = _build_lookup(B, NBITS, V, D)(bits_t, wflat)
    out = (
        out.reshape(D // 8, B // 128, 8, 128)
        .transpose(0, 2, 1, 3)
        .reshape(D, B)
        .T
    )
    return out
